# baseline (speedup 1.0000x reference)
import math

import jax
import jax.numpy as jnp
from jax.experimental import pallas as pl
from jax.experimental.pallas import tpu as pltpu

_T = 256
_D = 256
_N = _T + _D
_HALF = _T * _D
_LAM = 5.0
_SL = math.exp(-0.2 * 5.0)
_TI = 32
_NSTEPS = _T // _TI


def _dot(a, b, dims=(((1,), (0,)), ((), ()))):
    return jax.lax.dot_general(a, b, dims,
                               precision=jax.lax.Precision.HIGHEST,
                               preferred_element_type=jnp.float32)


def _dot16(a, b):
    return jax.lax.dot_general(a.astype(jnp.bfloat16), b.astype(jnp.bfloat16),
                               (((1,), (0,)), ((), ())),
                               preferred_element_type=jnp.float32)


def _relu(v):
    return jnp.maximum(v, 0.0)


def _body(x_ref, coords_ref, boxt_ref, boxdT_ref,
          W1_ref, b1_ref, W2_ref, b2_ref, Wa1_ref, Wg1_ref, Wme_ref,
          pea1_ref, pea2_ref,
          bias128_ref, W4_ref, Wepk_ref, bepk_ref, Wp1_ref, bp1_ref,
          Ge1_ref, Ge2_ref, Gp_ref, biasm1_ref, biasm2_ref,
          Wu_ref, bu_ref, wf1a_ref, wf1b_ref, bf1_ref, wf2_ref, bf2_ref,
          K_ref,
          embt_s, embd_s, Pt_s, Pd_s, Mt_s, Md_s, aggt_s, aggd_s, iou_s):
    i = pl.program_id(0)

    @pl.when(i == 0)
    def _k1():
        h = _relu(_dot(x_ref[...], W1_ref[...]) + b1_ref[...])
        emb = _dot(h, W2_ref[...]) + b2_ref[...]
        embt_s[...] = emb[:_T]
        embd_s[...] = emb[_T:]
        A = _dot(emb, Wa1_ref[:128, :])
        B = _dot(emb, Wa1_ref[128:, :])
        co = coords_ref[...]
        C = _dot(co, Wg1_ref[:4, :])
        Dm = _dot(co, Wg1_ref[4:, :])
        bf = jnp.bfloat16
        Pt_s[...] = jnp.concatenate([A[:_T], B[:_T], C[:_T], Dm[:_T]],
                                    axis=1).astype(bf)
        Pd_s[...] = (jnp.concatenate([B[_T:], A[_T:], Dm[_T:], C[_T:]], axis=1)
                     + bias128_ref[...]).astype(bf)
        M = _dot(emb, Wme_ref[...])
        Mt_s[...] = M[:_T] + biasm1_ref[...]
        Md_s[...] = M[_T:] + biasm2_ref[...]

    rows = _TI * _D
    trk = pl.ds(i * _TI, _TI)

    def rows_t(v):
        return jnp.broadcast_to(v[:, None, :], (_TI, _D, v.shape[-1])
                                ).reshape(rows, v.shape[-1])

    def rows_d(v):
        return jnp.broadcast_to(v[None, :, :], (_TI, _D, v.shape[-1])
                                ).reshape(rows, v.shape[-1])

    pre = _relu(rows_t(Pt_s[trk, :]) + rows_d(Pd_s[...]))
    xq = _dot16(pre, W4_ref[...])
    pre_e = _relu(_dot16(xq, Wepk_ref[...]) + bepk_ref[...])
    ph1 = _relu(_dot16(pea1_ref[...], Wp1_ref[...]) + bp1_ref[...])
    ph2 = _relu(_dot16(pea2_ref[...], Wp1_ref[...]) + bp1_ref[...])
    add1 = _dot16(pre_e, Ge1_ref[...]) + _dot16(ph1, Gp_ref[...])
    add2 = _dot16(pre_e, Ge2_ref[...]) + _dot16(ph2, Gp_ref[...])
    msg1 = _relu(rows_t(Mt_s[trk, :]) + add1)
    msg2 = _relu(rows_d(Md_s[...]) + add2)

    aggt_s[trk, :] = jnp.sum(msg2.reshape(_TI, _D, 128), axis=1)
    part = jnp.sum(msg1.reshape(_TI, _D, 128), axis=0)

    @pl.when(i == 0)
    def _():
        aggd_s[...] = part

    @pl.when(i > 0)
    def _():
        aggd_s[...] += part

    boxt = boxt_ref[trk, :]
    boxdT = boxdT_ref[...]
    tx1, ty1, tx2, ty2 = (boxt[:, k:k + 1] for k in range(4))
    dx1, dy1, dx2, dy2 = (boxdT[k:k + 1, :] for k in range(4))
    iw = _relu(jnp.minimum(tx2, dx2) - jnp.maximum(tx1, dx1))
    ih = _relu(jnp.minimum(ty2, dy2) - jnp.maximum(ty1, dy1))
    inter = iw * ih
    aa = (tx2 - tx1) * (ty2 - ty1)
    ab = (dx2 - dx1) * (dy2 - dy1)
    iou_s[trk, :] = inter / (aa + ab - inter + 1e-6)

    @pl.when(i == _NSTEPS - 1)
    def _k3():
        Wu_e = Wu_ref[:128, :]
        Wu_a = Wu_ref[128:, :]
        bu = bu_ref[...]
        ot = _relu(_dot(embt_s[...], Wu_e) + _dot(aggt_s[...], Wu_a) + bu)
        od = _relu(_dot(embd_s[...], Wu_e) + _dot(aggd_s[...], Wu_a) + bu)
        ns = jnp.sqrt(jnp.sum(ot * ot, axis=1, keepdims=True) + 1e-12)
        nd = jnp.sqrt(jnp.sum(od * od, axis=1, keepdims=True) + 1e-12)
        dots = _dot(ot, od, (((1,), (1,)), ((), ())))
        cos = dots / (ns * jnp.transpose(nd) + 1e-6)

        iou = iou_s[...]
        fin = jnp.full_like(cos, 0.0)
        for k in range(8):
            fin += wf2_ref[0, k] * _relu(cos * wf1a_ref[0, k]
                                         + iou * wf1b_ref[0, k]
                                         + bf1_ref[0, k])
        fin += bf2_ref[0, 0]

        K = jnp.exp(_LAM * fin)
        c = jnp.full((_T, 1), _SL, jnp.float32)
        r = jnp.full((1, _D), _SL, jnp.float32)
        s = jnp.float32(_SL)
        for _ in range(8):
            rs = jnp.sum(K, axis=1, keepdims=True) + c + 1e-9
            K = K / rs
            c = c / rs
            rr = jnp.sum(r) + s + 1e-9
            r = r / rr
            s = s / rr
            cs = jnp.sum(K, axis=0, keepdims=True) + r + 1e-9
            K = K / cs
            r = r / cs
            cc = jnp.sum(c) + s + 1e-9
            c = c / cc
            s = s / cc
        K_ref[...] = K


def kernel(x, coords_original, coords, edge_index, ground_truth,
           positional_edge_attr, frame, edges_number, track_num, det_num,
           W1, b1, W2, b2, Wa1, ba1, Wa2, ba2, Wg1, bg1, Wg2, bg2,
           We1, be1, We2, be2, Wp1, bp1, Wp2, bp2, Wm, bm, Wu, bu,
           Wf1, bf1, Wf2, bf2):
    f32 = jnp.float32
    bf16 = jnp.bfloat16
    row = lambda v: jnp.reshape(v, (1, -1)).astype(f32)

    z32 = jnp.zeros((32,), f32)
    W4 = jnp.stack([
        jnp.concatenate([Wa2[:, 0], z32, z32, z32]),
        jnp.concatenate([z32, Wa2[:, 0], z32, z32]),
        jnp.concatenate([z32, z32, Wg2[:, 0], z32]),
        jnp.concatenate([z32, z32, z32, Wg2[:, 0]]),
    ], axis=1)
    bias128 = row(jnp.concatenate([ba1, ba1, bg1, bg1]))
    z16 = jnp.zeros((16,), f32)
    Wepk = jnp.stack([
        jnp.concatenate([We1[0], z16]),
        jnp.concatenate([z16, We1[0]]),
        jnp.concatenate([We1[1], z16]),
        jnp.concatenate([z16, We1[1]]),
    ], axis=0)
    bq = jnp.stack([ba2[0], ba2[0], bg2[0], bg2[0]])[None, :]
    bepk = _dot(bq, Wepk) + row(jnp.concatenate([be1, be1]))
    wme, wmp, wmd = Wm[128], Wm[129], Wm[130]
    Ge = We2 * wme[None, :]
    Gz = jnp.zeros((16, 128), f32)
    Ge1 = jnp.concatenate([Ge, Gz], axis=0)
    Ge2 = jnp.concatenate([Gz, Ge], axis=0)
    Gp = Wp2 * wmp[None, :]
    biasm1 = row(be2[0] * wme + bp2[0] * wmp + wmd + bm)
    biasm2 = row(be2[0] * wme + bp2[0] * wmp - wmd + bm)

    fb = lambda shp: pl.BlockSpec(shp, lambda i: tuple(0 for _ in shp))
    eb1 = pl.BlockSpec((_TI * _D, 8), lambda i: (i, 0))
    eb2 = pl.BlockSpec((_TI * _D, 8), lambda i: (i + _NSTEPS, 0))
    scr = lambda shp: pltpu.VMEM(shp, f32)

    Kmat = pl.pallas_call(
        _body,
        grid=(_NSTEPS,),
        in_specs=[
            fb((_N, 512)), fb((_N, 4)), fb((_T, 4)), fb((4, _D)),
            fb((512, 256)), fb((1, 256)), fb((256, 128)), fb((1, 128)),
            fb((256, 32)), fb((8, 32)), fb((128, 128)),
            eb1, eb2,
            fb((1, 128)), fb((128, 4)), fb((4, 32)), fb((1, 32)),
            fb((8, 16)), fb((1, 16)),
            fb((32, 128)), fb((32, 128)), fb((16, 128)),
            fb((1, 128)), fb((1, 128)),
            fb((256, 128)), fb((1, 128)),
            fb((1, 8)), fb((1, 8)), fb((1, 8)), fb((1, 8)), fb((1, 1)),
        ],
        out_specs=pl.BlockSpec((_T, _D), lambda i: (0, 0)),
        out_shape=jax.ShapeDtypeStruct((_T, _D), f32),
        scratch_shapes=[
            scr((_T, 128)), scr((_T, 128)),
            pltpu.VMEM((_T, 128), jnp.bfloat16),
            pltpu.VMEM((_T, 128), jnp.bfloat16),
            scr((_T, 128)), scr((_T, 128)),
            scr((_T, 128)), scr((_D, 128)),
            scr((_T, _D)),
        ],
    )(x, coords, coords_original[:_T], jnp.transpose(coords_original[_T:]),
      W1, row(b1), W2, row(b2), Wa1, Wg1, Wm[:128],
      positional_edge_attr, positional_edge_attr,
      bias128, W4.astype(bf16), Wepk.astype(bf16), bepk,
      Wp1.astype(bf16), row(bp1),
      Ge1.astype(bf16), Ge2.astype(bf16), Gp.astype(bf16), biasm1, biasm2,
      Wu, row(bu), row(Wf1[0]), row(Wf1[1]), row(bf1), row(Wf2), row(bf2))

    norm = jnp.reshape(Kmat, (-1,))
    return (norm, norm, ground_truth, ground_truth,
            jnp.reshape(det_num, (1,)), jnp.reshape(track_num, (1,)))


# speedup vs baseline: 1.0696x; 1.0696x over previous
import math

import jax
import jax.numpy as jnp
from jax.experimental import pallas as pl
from jax.experimental.pallas import tpu as pltpu

_T = 256
_D = 256
_N = _T + _D
_HALF = _T * _D
_LAM = 5.0
_SL = math.exp(-0.2 * 5.0)
_TI = 32
_NSTEPS = _T // _TI


def _dot(a, b, dims=(((1,), (0,)), ((), ()))):
    return jax.lax.dot_general(a, b, dims,
                               precision=jax.lax.Precision.HIGHEST,
                               preferred_element_type=jnp.float32)


def _dot16(a, b):
    return jax.lax.dot_general(a.astype(jnp.bfloat16), b.astype(jnp.bfloat16),
                               (((1,), (0,)), ((), ())),
                               preferred_element_type=jnp.float32)


def _relu(v):
    return jnp.maximum(v, 0.0)


def _body(x_ref, coords_ref, boxt_ref, boxdT_ref,
          W1_ref, b1_ref, W2_ref, b2_ref, Wa1_ref, Wg1_ref, Wme_ref,
          pea1_ref, pea2_ref,
          Wa2_ref, Wg2_ref, ba1_ref, bg1_ref, ba2_ref, bg2_ref,
          We1_ref, be1_ref, We2_ref, be2_ref, Wp1_ref, bp1_ref,
          Wp2_ref, bp2_ref, Wmt_ref, bm_ref,
          Wu_ref, bu_ref, Wf1_ref, bf1_ref, Wf2_ref, bf2_ref,
          K_ref,
          embt_s, embd_s, Pt_s, Pd_s, Mt_s, Md_s, aggt_s, aggd_s, iou_s,
          w4_s, wepk_s, bepk_s, ge1_s, ge2_s, gp_s):
    i = pl.program_id(0)

    @pl.when(i == 0)
    def _k1():
        bf = jnp.bfloat16
        z32c = jnp.zeros((32, 1), jnp.float32)
        wa2 = Wa2_ref[...]
        wg2 = Wg2_ref[...]
        w4_s[...] = jnp.concatenate([
            jnp.concatenate([wa2, z32c, z32c, z32c], 0),
            jnp.concatenate([z32c, wa2, z32c, z32c], 0),
            jnp.concatenate([z32c, z32c, wg2, z32c], 0),
            jnp.concatenate([z32c, z32c, z32c, wg2], 0)], 1).astype(bf)
        z16r = jnp.zeros((1, 16), jnp.float32)
        we1a = We1_ref[0:1, :]
        we1b = We1_ref[1:2, :]
        wepk = jnp.concatenate([
            jnp.concatenate([we1a, z16r], 1),
            jnp.concatenate([z16r, we1a], 1),
            jnp.concatenate([we1b, z16r], 1),
            jnp.concatenate([z16r, we1b], 1)], 0)
        wepk_s[...] = wepk.astype(bf)
        ba2 = ba2_ref[...]
        bg2 = bg2_ref[...]
        bq = jnp.concatenate([ba2, ba2, bg2, bg2], 1)
        be1 = be1_ref[...]
        bepk_s[...] = _dot16(bq, wepk) + jnp.concatenate([be1, be1], 1)
        wme = Wmt_ref[0:1, :]
        wmp = Wmt_ref[1:2, :]
        wmd = Wmt_ref[2:3, :]
        ge = We2_ref[...] * wme
        zg = jnp.zeros((16, 128), jnp.float32)
        ge1_s[...] = jnp.concatenate([ge, zg], 0).astype(bf)
        ge2_s[...] = jnp.concatenate([zg, ge], 0).astype(bf)
        gp_s[...] = (Wp2_ref[...] * wmp).astype(bf)

        h = _relu(_dot(x_ref[...], W1_ref[...]) + b1_ref[...])
        emb = _dot(h, W2_ref[...]) + b2_ref[...]
        embt_s[...] = emb[:_T]
        embd_s[...] = emb[_T:]
        A = _dot(emb, Wa1_ref[:128, :])
        B = _dot(emb, Wa1_ref[128:, :])
        co = coords_ref[...]
        C = _dot(co, Wg1_ref[:4, :])
        Dm = _dot(co, Wg1_ref[4:, :])
        bias128 = jnp.concatenate([ba1_ref[...], ba1_ref[...],
                                   bg1_ref[...], bg1_ref[...]], 1)
        Pt_s[...] = jnp.concatenate([A[:_T], B[:_T], C[:_T], Dm[:_T]],
                                    axis=1).astype(bf)
        Pd_s[...] = (jnp.concatenate([B[_T:], A[_T:], Dm[_T:], C[_T:]], axis=1)
                     + bias128).astype(bf)
        bmsg = be2_ref[0, 0] * wme + bp2_ref[0, 0] * wmp + bm_ref[...]
        M = _dot(emb, Wme_ref[...])
        Mt_s[...] = M[:_T] + (bmsg + wmd)
        Md_s[...] = M[_T:] + (bmsg - wmd)

    rows = _TI * _D
    trk = pl.ds(i * _TI, _TI)

    def rows_t(v):
        return jnp.broadcast_to(v[:, None, :], (_TI, _D, v.shape[-1])
                                ).reshape(rows, v.shape[-1])

    def rows_d(v):
        return jnp.broadcast_to(v[None, :, :], (_TI, _D, v.shape[-1])
                                ).reshape(rows, v.shape[-1])

    pre = _relu(rows_t(Pt_s[trk, :]) + rows_d(Pd_s[...]))
    xq = _dot16(pre, w4_s[...])
    pre_e = _relu(_dot16(xq, wepk_s[...]) + bepk_s[...])
    ph1 = _relu(_dot16(pea1_ref[...], Wp1_ref[...]) + bp1_ref[...])
    ph2 = _relu(_dot16(pea2_ref[...], Wp1_ref[...]) + bp1_ref[...])
    add1 = _dot16(pre_e, ge1_s[...]) + _dot16(ph1, gp_s[...])
    add2 = _dot16(pre_e, ge2_s[...]) + _dot16(ph2, gp_s[...])
    msg1 = _relu(rows_t(Mt_s[trk, :]) + add1)
    msg2 = _relu(rows_d(Md_s[...]) + add2)

    aggt_s[trk, :] = jnp.sum(msg2.reshape(_TI, _D, 128), axis=1)
    part = jnp.sum(msg1.reshape(_TI, _D, 128), axis=0)

    @pl.when(i == 0)
    def _():
        aggd_s[...] = part

    @pl.when(i > 0)
    def _():
        aggd_s[...] += part

    boxt = boxt_ref[trk, :]
    boxdT = boxdT_ref[...]
    tx1, ty1, tx2, ty2 = (boxt[:, k:k + 1] for k in range(4))
    dx1, dy1, dx2, dy2 = (boxdT[k:k + 1, :] for k in range(4))
    iw = _relu(jnp.minimum(tx2, dx2) - jnp.maximum(tx1, dx1))
    ih = _relu(jnp.minimum(ty2, dy2) - jnp.maximum(ty1, dy1))
    inter = iw * ih
    aa = (tx2 - tx1) * (ty2 - ty1)
    ab = (dx2 - dx1) * (dy2 - dy1)
    iou_s[trk, :] = inter / (aa + ab - inter + 1e-6)

    @pl.when(i == _NSTEPS - 1)
    def _k3():
        Wu_e = Wu_ref[:128, :]
        Wu_a = Wu_ref[128:, :]
        bu = bu_ref[...]
        ot = _relu(_dot(embt_s[...], Wu_e) + _dot(aggt_s[...], Wu_a) + bu)
        od = _relu(_dot(embd_s[...], Wu_e) + _dot(aggd_s[...], Wu_a) + bu)
        ns = jnp.sqrt(jnp.sum(ot * ot, axis=1, keepdims=True) + 1e-12)
        nd = jnp.sqrt(jnp.sum(od * od, axis=1, keepdims=True) + 1e-12)
        dots = _dot(ot, od, (((1,), (1,)), ((), ())))
        cos = dots / (ns * jnp.transpose(nd) + 1e-6)

        iou = iou_s[...]
        fin = jnp.full_like(cos, 0.0)
        for k in range(8):
            fin += Wf2_ref[k, 0] * _relu(cos * Wf1_ref[0, k]
                                         + iou * Wf1_ref[1, k]
                                         + bf1_ref[0, k])
        fin += bf2_ref[0, 0]

        K = jnp.exp(_LAM * fin)
        c = jnp.full((_T, 1), _SL, jnp.float32)
        r = jnp.full((1, _D), _SL, jnp.float32)
        s = jnp.float32(_SL)
        for _ in range(8):
            rs = jnp.sum(K, axis=1, keepdims=True) + c + 1e-9
            K = K / rs
            c = c / rs
            rr = jnp.sum(r) + s + 1e-9
            r = r / rr
            s = s / rr
            cs = jnp.sum(K, axis=0, keepdims=True) + r + 1e-9
            K = K / cs
            r = r / cs
            cc = jnp.sum(c) + s + 1e-9
            c = c / cc
            s = s / cc
        K_ref[...] = K


def kernel(x, coords_original, coords, edge_index, ground_truth,
           positional_edge_attr, frame, edges_number, track_num, det_num,
           W1, b1, W2, b2, Wa1, ba1, Wa2, ba2, Wg1, bg1, Wg2, bg2,
           We1, be1, We2, be2, Wp1, bp1, Wp2, bp2, Wm, bm, Wu, bu,
           Wf1, bf1, Wf2, bf2):
    f32 = jnp.float32
    row = lambda v: jnp.reshape(v, (1, -1)).astype(f32)

    fb = lambda shp: pl.BlockSpec(shp, lambda i: tuple(0 for _ in shp))
    eb1 = pl.BlockSpec((_TI * _D, 8), lambda i: (i, 0))
    eb2 = pl.BlockSpec((_TI * _D, 8), lambda i: (i + _NSTEPS, 0))
    scr = lambda shp: pltpu.VMEM(shp, f32)

    Kmat = pl.pallas_call(
        _body,
        grid=(_NSTEPS,),
        in_specs=[
            fb((_N, 512)), fb((_N, 4)), fb((_T, 4)), fb((4, _D)),
            fb((512, 256)), fb((1, 256)), fb((256, 128)), fb((1, 128)),
            fb((256, 32)), fb((8, 32)), fb((128, 128)),
            eb1, eb2,
            fb((32, 1)), fb((32, 1)), fb((1, 32)), fb((1, 32)),
            fb((1, 1)), fb((1, 1)),
            fb((2, 16)), fb((1, 16)), fb((16, 1)), fb((1, 1)),
            fb((8, 16)), fb((1, 16)), fb((16, 1)), fb((1, 1)),
            fb((3, 128)), fb((1, 128)),
            fb((256, 128)), fb((1, 128)),
            fb((2, 8)), fb((1, 8)), fb((8, 1)), fb((1, 1)),
        ],
        out_specs=pl.BlockSpec((_T, _D), lambda i: (0, 0)),
        out_shape=jax.ShapeDtypeStruct((_T, _D), f32),
        scratch_shapes=[
            scr((_T, 128)), scr((_T, 128)),
            pltpu.VMEM((_T, 128), jnp.bfloat16),
            pltpu.VMEM((_T, 128), jnp.bfloat16),
            scr((_T, 128)), scr((_T, 128)),
            scr((_T, 128)), scr((_D, 128)),
            scr((_T, _D)),
            pltpu.VMEM((128, 4), jnp.bfloat16),
            pltpu.VMEM((4, 32), jnp.bfloat16),
            scr((1, 32)),
            pltpu.VMEM((32, 128), jnp.bfloat16),
            pltpu.VMEM((32, 128), jnp.bfloat16),
            pltpu.VMEM((16, 128), jnp.bfloat16),
        ],
    )(x, coords, coords_original[:_T], jnp.transpose(coords_original[_T:]),
      W1, row(b1), W2, row(b2), Wa1, Wg1, Wm[:128],
      positional_edge_attr, positional_edge_attr,
      Wa2, Wg2, row(ba1), row(bg1), row(ba2), row(bg2),
      We1, row(be1), We2, row(be2), Wp1, row(bp1), Wp2, row(bp2),
      Wm[128:131], row(bm),
      Wu, row(bu), Wf1, row(bf1), Wf2, row(bf2))

    norm = jnp.reshape(Kmat, (-1,))
    return (norm, norm, ground_truth, ground_truth,
            jnp.reshape(det_num, (1,)), jnp.reshape(track_num, (1,)))
